# baseline (speedup 1.0000x reference)
import functools

import jax
import jax.numpy as jnp
from jax import lax
from jax.experimental import pallas as pl
from jax.experimental.pallas import tpu as pltpu
from jax.experimental.pallas import tpu_sc as plsc

_FIELD_DIM = 100000
_NUM_F = 26
_EMBED_D = 16
_BATCH = 16384

_NW = 32
_SAMPLES_PER_W = _BATCH // _NW
_G = 16
_NGROUP = _SAMPLES_PER_W // _G
_GELEM = _G * _NUM_F
_DMA_PIECES = [(0, 128), (128, 128), (256, 128), (384, 32)]


def _fm_kernel(x_hbm, wemb_hbm, wlin_hbm, bias_hbm, out_hbm,
               xst_v, slots_v, xis_v, cols_v, eb_v, lin_v, outb_v, bias_v,
               sem_e, sem_l):
    wid = lax.axis_index("s") * 2 + lax.axis_index("c")

    pltpu.sync_copy(bias_hbm, bias_v)
    bias_vec = bias_v[pl.ds(0, 16)]
    iota = lax.iota(jnp.int32, 16)

    def group_body(k, carry):
        base = (wid * _NGROUP + k) * _GELEM
        pltpu.sync_copy(x_hbm.at[pl.ds(base, _GELEM)], xst_v)

        for f in range(_NUM_F):
            xi = plsc.load_gather(xst_v, [iota * _NUM_F + f]) + (f * _FIELD_DIM - 1)
            slots_v[pl.ds(f * 16, 16)] = lax.shift_right_arithmetic(xi, 3)
            cols_v[pl.ds(f * 16, 16)] = lax.shift_left(jnp.bitwise_and(xi, 7), 4)
            xis_v[pl.ds(f * 16, 16)] = xi

        handles = []
        for (o, n) in _DMA_PIECES:
            handles.append(pltpu.async_copy(
                wemb_hbm.at[slots_v.at[pl.ds(o, n)]], eb_v.at[pl.ds(o, n)], sem_e))
            handles.append(pltpu.async_copy(
                wlin_hbm.at[xis_v.at[pl.ds(o, n)]], lin_v.at[pl.ds(o, n)], sem_l))
        for h in handles:
            h.wait()

        s = [jnp.zeros((16,), jnp.float32) for _ in range(_EMBED_D)]
        sq = [jnp.zeros((16,), jnp.float32) for _ in range(_EMBED_D)]
        for f in range(_NUM_F):
            rowv = iota + f * 16
            colb = cols_v[pl.ds(f * 16, 16)]
            for d in range(_EMBED_D):
                g = plsc.load_gather(eb_v, [rowv, colb + d])
                s[d] = s[d] + g
                sq[d] = sq[d] + g * g
        acc = jnp.zeros((16,), jnp.float32)
        for d in range(_EMBED_D):
            acc = acc + (s[d] * s[d] - sq[d])

        lacc = jnp.zeros((16,), jnp.float32)
        for f in range(_NUM_F):
            lacc = lacc + lin_v[pl.ds(f * 16, 16)]

        z = lacc + bias_vec + 0.5 * acc
        outb_v[...] = 1.0 / (1.0 + jnp.exp(-z))
        pltpu.sync_copy(outb_v, out_hbm.at[pl.ds(wid * _SAMPLES_PER_W + k * _G, _G)])
        return carry

    lax.fori_loop(0, _NGROUP, group_body, 0)


def kernel(x, W_emb, W_lin, bias):
    x1d = x.astype(jnp.int32).reshape(-1)
    wemb128 = W_emb.reshape(-1, 128)
    wlin1d = W_lin.reshape(-1)

    mesh = plsc.VectorSubcoreMesh(core_axis_name="c", subcore_axis_name="s")
    run = functools.partial(
        pl.kernel,
        mesh=mesh,
        compiler_params=pltpu.CompilerParams(needs_layout_passes=False),
        out_type=jax.ShapeDtypeStruct((_BATCH,), jnp.float32),
        scratch_types=[
            pltpu.VMEM((_GELEM,), jnp.int32),
            pltpu.VMEM((_GELEM,), jnp.int32),
            pltpu.VMEM((_GELEM,), jnp.int32),
            pltpu.VMEM((_GELEM,), jnp.int32),
            pltpu.VMEM((_GELEM, 128), jnp.float32),
            pltpu.VMEM((_GELEM,), jnp.float32),
            pltpu.VMEM((_G,), jnp.float32),
            pltpu.VMEM((16,), jnp.float32),
            pltpu.SemaphoreType.DMA,
            pltpu.SemaphoreType.DMA,
        ],
    )(_fm_kernel)
    return run(x1d, wemb128, wlin1d, jnp.broadcast_to(bias, (16,)))


# speedup vs baseline: 1.0870x; 1.0870x over previous
import functools

import jax
import jax.numpy as jnp
from jax import lax
from jax.experimental import pallas as pl
from jax.experimental.pallas import tpu as pltpu
from jax.experimental.pallas import tpu_sc as plsc

_FIELD_DIM = 100000
_NUM_F = 26
_EMBED_D = 16
_BATCH = 16384

_NW = 32
_SAMPLES_PER_W = _BATCH // _NW
_CHUNK = 64
_NCHUNK = _SAMPLES_PER_W // _CHUNK
_CELEM = _CHUNK * _NUM_F
_NROW = _CELEM // 128
_NGROUP = _CHUNK // 16


def _fm_kernel(xt_hbm, wemb_hbm, wlin_hbm, bias_hbm, out_hbm,
               xst_v, idx_v, rows_v, lin_v, tb_v, outb_v, bias_v,
               sem_e, sem_l):
    wid = lax.axis_index("s") * 2 + lax.axis_index("c")

    pltpu.sync_copy(bias_hbm, bias_v)
    bias_vec = bias_v[pl.ds(0, 16)]
    iota = lax.iota(jnp.int32, 16)

    def chunk_body(k, carry):
        s0 = wid * _SAMPLES_PER_W + k * _CHUNK
        pltpu.sync_copy(xt_hbm.at[:, pl.ds(s0, _CHUNK)], xst_v)

        for f in range(_NUM_F):
            off = f * _FIELD_DIM - 1
            for sb in range(_CHUNK // 16):
                xi = xst_v[f, pl.ds(sb * 16, 16)] + off
                plsc.store_scatter(
                    idx_v, [(sb * 16 + iota) * _NUM_F + f], xi)

        handles = []
        for j in range(_NROW):
            piece = pl.ds(j * 128, 128)
            handles.append(pltpu.async_copy(
                wemb_hbm.at[idx_v.at[piece]], rows_v.at[piece], sem_e))
            handles.append(pltpu.async_copy(
                wlin_hbm.at[idx_v.at[piece]], lin_v.at[piece], sem_l))
        for h in handles:
            h.wait()

        for g in range(_NGROUP):
            def sample_body(c, carry2):
                r0 = (g * 16 + c) * _NUM_F
                s = jnp.zeros((16,), jnp.float32)
                sq = jnp.zeros((16,), jnp.float32)
                for f in range(_NUM_F):
                    r = rows_v[r0 + f, :]
                    s = s + r
                    sq = sq + r * r
                t = s * s - sq
                plsc.store_scatter(tb_v, [iota * 16 + c], t)
                return carry2
            lax.fori_loop(0, 16, sample_body, 0)

            acc = jnp.zeros((16,), jnp.float32)
            for d in range(16):
                acc = acc + tb_v[pl.ds(d * 16, 16)]

            lbase = g * 16 * _NUM_F
            lacc = jnp.zeros((16,), jnp.float32)
            for f in range(_NUM_F):
                lacc = lacc + plsc.load_gather(lin_v, [iota * _NUM_F + (lbase + f)])

            z = lacc + bias_vec + 0.5 * acc
            outb_v[pl.ds(g * 16, 16)] = 1.0 / (1.0 + jnp.exp(-z))

        pltpu.sync_copy(outb_v, out_hbm.at[pl.ds(s0, _CHUNK)])
        return carry

    lax.fori_loop(0, _NCHUNK, chunk_body, 0)


def kernel(x, W_emb, W_lin, bias):
    xt = x.astype(jnp.int32).T
    wlin1d = W_lin.reshape(-1)

    mesh = plsc.VectorSubcoreMesh(core_axis_name="c", subcore_axis_name="s")
    run = functools.partial(
        pl.kernel,
        mesh=mesh,
        compiler_params=pltpu.CompilerParams(
            needs_layout_passes=False, use_tc_tiling_on_sc=False),
        out_type=jax.ShapeDtypeStruct((_BATCH,), jnp.float32),
        scratch_types=[
            pltpu.VMEM((_NUM_F, _CHUNK), jnp.int32),
            pltpu.VMEM((_CELEM,), jnp.int32),
            pltpu.VMEM((_CELEM, _EMBED_D), jnp.float32),
            pltpu.VMEM((_CELEM,), jnp.float32),
            pltpu.VMEM((256,), jnp.float32),
            pltpu.VMEM((_CHUNK,), jnp.float32),
            pltpu.VMEM((16,), jnp.float32),
            pltpu.SemaphoreType.DMA,
            pltpu.SemaphoreType.DMA,
        ],
    )(_fm_kernel)
    return run(xt, W_emb, wlin1d, jnp.broadcast_to(bias, (16,)))
